# baseline (speedup 1.0000x reference)
import jax
import jax.numpy as jnp
from jax import lax
from jax.experimental import pallas as pl
from jax.experimental.pallas import tpu as pltpu
from jax.experimental.pallas import tpu_sc as plsc

NUM_ROWS = 16384
NUM_COLS = 1000

_info = plsc.get_sparse_core_info()
NC = _info.num_cores
NS = _info.num_subcores
L = _info.num_lanes
NW = NC * NS

ROWS_PER_W = NUM_ROWS // NW
CHUNK_ROWS = 64
NCHUNK = ROWS_PER_W // CHUNK_ROWS
CHUNK_WORDS = CHUNK_ROWS * NUM_COLS
GROUPS = CHUNK_ROWS // L


def _body(x_hbm, out_hbm, buf0, buf1, idx_v, sem0, sem1):
    cid = lax.axis_index("c")
    sid = lax.axis_index("s")
    wid = sid * NC + cid
    base_row = wid * ROWS_PER_W

    zeros16 = jnp.zeros((L,), jnp.float32)
    ones16 = jnp.ones((L,), jnp.float32)
    iota16 = lax.iota(jnp.int32, L)

    def _zero(j, carry):
        buf0[pl.ds(j * L, L)] = zeros16
        buf1[pl.ds(j * L, L)] = zeros16
        return carry

    lax.fori_loop(0, CHUNK_WORDS // L, _zero, 0)

    bufs = (buf0, buf1)
    sems = (sem0, sem1)
    handles = [None, None]
    old_cols = [None, None]

    for k in range(NCHUNK):
        slot = k % 2
        buf = bufs[slot]
        row0 = base_row + k * CHUNK_ROWS

        if handles[slot] is not None:
            handles[slot].wait()
            for g in range(GROUPS):
                colv = old_cols[slot][g]
                flat = (iota16 + g * L) * NUM_COLS + colv
                plsc.store_scatter(buf, [flat], zeros16,
                                   mask=colv < NUM_COLS)

        pltpu.sync_copy(x_hbm.at[pl.ds(row0, CHUNK_ROWS)], idx_v)
        cols = []
        for g in range(GROUPS):
            colv = idx_v[pl.ds(g * L, L)]
            flat = (iota16 + g * L) * NUM_COLS + colv
            plsc.store_scatter(buf, [flat], ones16,
                               mask=colv < NUM_COLS)
            cols.append(colv)
        old_cols[slot] = cols

        handles[slot] = pltpu.async_copy(
            buf, out_hbm.at[pl.ds(row0 * NUM_COLS, CHUNK_WORDS)],
            sems[slot])

    handles[0].wait()
    handles[1].wait()


@jax.jit
def kernel(x):
    mesh = plsc.VectorSubcoreMesh(core_axis_name="c", subcore_axis_name="s")
    out = pl.kernel(
        _body,
        out_type=jax.ShapeDtypeStruct((NUM_ROWS * NUM_COLS,), jnp.float32),
        mesh=mesh,
        compiler_params=pltpu.CompilerParams(needs_layout_passes=False),
        scratch_types=[
            pltpu.VMEM((CHUNK_WORDS,), jnp.float32),
            pltpu.VMEM((CHUNK_WORDS,), jnp.float32),
            pltpu.VMEM((CHUNK_ROWS,), jnp.int32),
            pltpu.SemaphoreType.DMA,
            pltpu.SemaphoreType.DMA,
        ],
    )(x.astype(jnp.int32))
    return out.reshape(NUM_ROWS, NUM_COLS)


# speedup vs baseline: 1.7477x; 1.7477x over previous
import jax
import jax.numpy as jnp
from jax import lax
from jax.experimental import pallas as pl
from jax.experimental.pallas import tpu as pltpu
from jax.experimental.pallas import tpu_sc as plsc

NUM_ROWS = 16384
NUM_COLS = 1000

_info = plsc.get_sparse_core_info()
NC = _info.num_cores
NS = _info.num_subcores
L = _info.num_lanes
NW = NC * NS

ROWS_PER_W = NUM_ROWS // NW
CHUNK_ROWS = 32
NCHUNK = ROWS_PER_W // CHUNK_ROWS
GROUPS = CHUNK_ROWS // L
_FILL_STARTS = tuple(range(0, NUM_COLS - L, L)) + (NUM_COLS - L,)


def _body(x_hbm, out_hbm, buf0, buf1, idx_v, sem0, sem1):
    cid = lax.axis_index("c")
    sid = lax.axis_index("s")
    wid = sid * NC + cid
    base_row = wid * ROWS_PER_W

    zeros16 = jnp.zeros((L,), jnp.float32)
    ones16 = jnp.ones((L,), jnp.float32)
    iota16 = lax.iota(jnp.int32, L)

    def _zero(r, carry):
        for c0 in _FILL_STARTS:
            buf0[r, pl.ds(c0, L)] = zeros16
            buf1[r, pl.ds(c0, L)] = zeros16
        return carry

    lax.fori_loop(0, CHUNK_ROWS, _zero, 0)

    bufs = (buf0, buf1)
    sems = (sem0, sem1)
    handles = [None, None]
    old_cols = [None, None]

    for k in range(NCHUNK):
        slot = k % 2
        buf = bufs[slot]
        row0 = base_row + k * CHUNK_ROWS

        if handles[slot] is not None:
            handles[slot].wait()
            for g in range(GROUPS):
                colv = old_cols[slot][g]
                plsc.store_scatter(buf, [iota16 + g * L, colv], zeros16,
                                   mask=colv < NUM_COLS)

        pltpu.sync_copy(x_hbm.at[pl.ds(row0, CHUNK_ROWS)], idx_v)
        cols = []
        for g in range(GROUPS):
            colv = idx_v[pl.ds(g * L, L)]
            plsc.store_scatter(buf, [iota16 + g * L, colv], ones16,
                               mask=colv < NUM_COLS)
            cols.append(colv)
        old_cols[slot] = cols

        handles[slot] = pltpu.async_copy(
            buf, out_hbm.at[pl.ds(row0, CHUNK_ROWS)], sems[slot])

    handles[0].wait()
    handles[1].wait()


@jax.jit
def kernel(x):
    mesh = plsc.VectorSubcoreMesh(core_axis_name="c", subcore_axis_name="s")
    return pl.kernel(
        _body,
        out_type=jax.ShapeDtypeStruct((NUM_ROWS, NUM_COLS), jnp.float32),
        mesh=mesh,
        compiler_params=pltpu.CompilerParams(needs_layout_passes=False),
        scratch_types=[
            pltpu.VMEM((CHUNK_ROWS, NUM_COLS), jnp.float32),
            pltpu.VMEM((CHUNK_ROWS, NUM_COLS), jnp.float32),
            pltpu.VMEM((CHUNK_ROWS,), jnp.int32),
            pltpu.SemaphoreType.DMA,
            pltpu.SemaphoreType.DMA,
        ],
    )(x.astype(jnp.int32))
